# baseline (speedup 1.0000x reference)
import jax
import jax.numpy as jnp
from jax import lax
from jax.experimental import pallas as pl
from jax.experimental.pallas import tpu as pltpu
from jax.experimental.pallas import tpu_sc as plsc

NC = 2
NS = 16
NW = NC * NS
L = 16

_B, _S, _D = 1024, 200, 64
_DP = 2 * _D
_ROWS_W = _B // NW


_V = 1000000
_NBLK = _V // 128
_TAIL = _V - _NBLK * 128
_BLK_W = _NBLK // NW + 1


def _make_transpose():
    mesh = plsc.VectorSubcoreMesh(
        core_axis_name="c", subcore_axis_name="s",
        num_cores=NC, num_subcores=NS)

    @pl.kernel(
        out_type=jax.ShapeDtypeStruct((_V, _DP), jnp.float32),
        mesh=mesh,
        compiler_params=pltpu.CompilerParams(
            use_tc_tiling_on_sc=True, needs_layout_passes=False),
        scratch_types=[
            pltpu.VMEM((_D, _DP), jnp.float32),
            pltpu.VMEM((_DP, _DP), jnp.float32),
            pltpu.SemaphoreType.DMA,
            pltpu.SemaphoreType.DMA,
        ],
    )
    def k(tt_hbm, out_hbm, blk_v, rows_v, gsem, psem):
        wid = lax.axis_index("s") * NC + lax.axis_index("c")
        jota = lax.iota(jnp.int32, L)

        def do_block(c, width):
            pltpu.async_copy(
                tt_hbm.at[:, pl.ds(c * _DP, width)],
                blk_v.at[:, pl.ds(0, width)], gsem).wait()

            def trow(t, carry):
                for j0 in range(_D // L):
                    vals = plsc.load_gather(
                        blk_v, [j0 * L + jota, jnp.full((L,), 0, jnp.int32) + t])
                    rows_v[t, pl.ds(j0 * L, L)] = vals
                return carry

            lax.fori_loop(0, width, trow, 0)
            pltpu.async_copy(
                rows_v.at[pl.ds(0, width)],
                out_hbm.at[pl.ds(c * _DP, width)], psem).wait()

        def blk_loop(t, carry):
            c = wid + t * NW

            @pl.when(c < _NBLK)
            def _():
                do_block(c, _DP)
            return carry

        lax.fori_loop(0, _BLK_W, blk_loop, 0)

    return k


def _make_kernel():
    mesh = plsc.VectorSubcoreMesh(
        core_axis_name="c", subcore_axis_name="s",
        num_cores=NC, num_subcores=NS)

    @pl.kernel(
        out_type=jax.ShapeDtypeStruct((_B, _S, _D), jnp.float32),
        mesh=mesh,
        compiler_params=pltpu.CompilerParams(use_tc_tiling_on_sc=True),
        scratch_types=[
            pltpu.VMEM((_ROWS_W * _S,), jnp.int32),
            pltpu.VMEM((_S, _D), jnp.float32),
            pltpu.VMEM((_S, _DP), jnp.float32),
            pltpu.VMEM((_S, _D), jnp.float32),
            pltpu.SemaphoreType.DMA,
        ],
    )
    def k(x_hbm, table_hbm, pe_hbm, out_hbm, idx_v, pe_v, rows_v, sum_v, sem):
        wid = lax.axis_index("s") * NC + lax.axis_index("c")
        base = wid * _ROWS_W
        pltpu.sync_copy(x_hbm.at[pl.ds(base * _S, _ROWS_W * _S)], idx_v)
        pltpu.sync_copy(pe_hbm.at[pl.ds(0, _S)], pe_v)

        def chunk_body(kk, carry):
            pltpu.async_copy(
                table_hbm.at[idx_v.at[pl.ds(kk * _S, _S)]], rows_v, sem
            ).wait()

            def row_body(r, c2):
                for c in range(_D // L):
                    sl = pl.ds(c * L, L)
                    sum_v[r, sl] = rows_v[r, sl] + pe_v[r, sl]
                return c2

            lax.fori_loop(0, _S, row_body, 0)
            pltpu.sync_copy(sum_v, out_hbm.at[base + kk])
            return carry

        lax.fori_loop(0, _ROWS_W, chunk_body, 0)

    return k


_transpose_call = _make_transpose()
_kernel_call = _make_kernel()


def kernel(x, token_table, pe):
    tab128 = _transpose_call(jnp.transpose(token_table))
    tail = jnp.pad(token_table[_NBLK * 128:], ((0, 0), (0, _D)))
    tab128 = lax.dynamic_update_slice(tab128, tail, (_NBLK * 128, 0))
    return _kernel_call(x.reshape(-1), tab128, pe)


# speedup vs baseline: 1.5074x; 1.5074x over previous
import jax
import jax.numpy as jnp
from jax import lax
from jax.experimental import pallas as pl
from jax.experimental.pallas import tpu as pltpu
from jax.experimental.pallas import tpu_sc as plsc

NC = 2
NS = 16
NW = NC * NS
L = 16

_B, _S, _D = 1024, 200, 64
_DP = 2 * _D
_ROWS_W = _B // NW


_V = 1000000
_NBLK = _V // 128
_TAIL = _V - _NBLK * 128
_BLK_W = _NBLK // NW + 1


def _make_transpose():
    mesh = plsc.VectorSubcoreMesh(
        core_axis_name="c", subcore_axis_name="s",
        num_cores=NC, num_subcores=NS)

    @pl.kernel(
        out_type=jax.ShapeDtypeStruct((_V, _DP), jnp.float32),
        mesh=mesh,
        compiler_params=pltpu.CompilerParams(
            use_tc_tiling_on_sc=True, needs_layout_passes=False),
        scratch_types=[
            pltpu.VMEM((2, _D, _DP), jnp.float32),
            pltpu.VMEM((2, _DP, _DP), jnp.float32),
            pltpu.SemaphoreType.DMA,
            pltpu.SemaphoreType.DMA,
        ],
    )
    def k(tt_hbm, out_hbm, blk_v, rows_v, gsem, psem):
        wid = lax.axis_index("s") * NC + lax.axis_index("c")
        jota = lax.iota(jnp.int32, L)
        tvecs = [c16 * L + jota for c16 in range(_DP // L)]

        def blk_loop(t, carry):
            c = wid + t * NW

            @pl.when(c < _NBLK)
            def _():
                p = t % 2
                pltpu.make_async_copy(
                    tt_hbm.at[:, pl.ds(0, _DP)], blk_v.at[p], gsem).wait()
                cn = c + NW

                @pl.when(cn < _NBLK)
                def _():
                    pltpu.async_copy(
                        tt_hbm.at[:, pl.ds(cn * _DP, _DP)],
                        blk_v.at[1 - p], gsem)

                @pl.when(t >= 2)
                def _():
                    pltpu.make_async_copy(
                        rows_v.at[0], out_hbm.at[pl.ds(0, _DP)], psem).wait()

                def jrow(j2, carry2):
                    for dj in range(2):
                        j = j2 * 2 + dj
                        jvec = jota * 0 + j
                        for c16 in range(_DP // L):
                            vals = blk_v[p, j, pl.ds(c16 * L, L)]
                            plsc.store_scatter(
                                rows_v.at[p], [tvecs[c16], jvec], vals)
                    return carry2

                lax.fori_loop(0, _D // 2, jrow, 0)
                pltpu.async_copy(
                    rows_v.at[p], out_hbm.at[pl.ds(c * _DP, _DP)], psem)
            return carry

        pltpu.async_copy(
            tt_hbm.at[:, pl.ds(wid * _DP, _DP)], blk_v.at[0], gsem)
        lax.fori_loop(0, _BLK_W, blk_loop, 0)
        for _ in range(2):
            pltpu.make_async_copy(
                rows_v.at[0], out_hbm.at[pl.ds(0, _DP)], psem).wait()

    return k


def _make_kernel():
    mesh = plsc.VectorSubcoreMesh(
        core_axis_name="c", subcore_axis_name="s",
        num_cores=NC, num_subcores=NS)

    @pl.kernel(
        out_type=jax.ShapeDtypeStruct((_B, _S, _D), jnp.float32),
        mesh=mesh,
        compiler_params=pltpu.CompilerParams(use_tc_tiling_on_sc=True),
        scratch_types=[
            pltpu.VMEM((_ROWS_W * _S,), jnp.int32),
            pltpu.VMEM((_S, _D), jnp.float32),
            pltpu.VMEM((_S, _DP), jnp.float32),
            pltpu.VMEM((_S, _D), jnp.float32),
            pltpu.SemaphoreType.DMA,
        ],
    )
    def k(x_hbm, table_hbm, pe_hbm, out_hbm, idx_v, pe_v, rows_v, sum_v, sem):
        wid = lax.axis_index("s") * NC + lax.axis_index("c")
        base = wid * _ROWS_W
        pltpu.sync_copy(x_hbm.at[pl.ds(base * _S, _ROWS_W * _S)], idx_v)
        pltpu.sync_copy(pe_hbm.at[pl.ds(0, _S)], pe_v)

        def chunk_body(kk, carry):
            pltpu.async_copy(
                table_hbm.at[idx_v.at[pl.ds(kk * _S, _S)]], rows_v, sem
            ).wait()

            def row_body(r, c2):
                for c in range(_D // L):
                    sl = pl.ds(c * L, L)
                    sum_v[r, sl] = rows_v[r, sl] + pe_v[r, sl]
                return c2

            lax.fori_loop(0, _S, row_body, 0)
            pltpu.sync_copy(sum_v, out_hbm.at[base + kk])
            return carry

        lax.fori_loop(0, _ROWS_W, chunk_body, 0)

    return k


_transpose_call = _make_transpose()
_kernel_call = _make_kernel()


def kernel(x, token_table, pe):
    tab128 = _transpose_call(jnp.transpose(token_table))
    tail = jnp.pad(token_table[_NBLK * 128:], ((0, 0), (0, _D)))
    tab128 = lax.dynamic_update_slice(tab128, tail, (_NBLK * 128, 0))
    return _kernel_call(x.reshape(-1), tab128, pe)


# speedup vs baseline: 2.8033x; 1.8597x over previous
import jax
import jax.numpy as jnp
from jax import lax
from jax.experimental import pallas as pl
from jax.experimental.pallas import tpu as pltpu
from jax.experimental.pallas import tpu_sc as plsc

NC = 2
NS = 16
NW = NC * NS
L = 16

_B, _S, _D = 1024, 200, 64
_DP = 2 * _D
_ROWS_W = _B // NW


def _make_kernel():
    mesh = plsc.VectorSubcoreMesh(
        core_axis_name="c", subcore_axis_name="s",
        num_cores=NC, num_subcores=NS)

    @pl.kernel(
        out_type=jax.ShapeDtypeStruct((_B, _S, _D), jnp.float32),
        mesh=mesh,
        compiler_params=pltpu.CompilerParams(use_tc_tiling_on_sc=True),
        scratch_types=[
            pltpu.VMEM((2 * _S,), jnp.int32),
            pltpu.VMEM((_S, _D), jnp.float32),
            pltpu.VMEM((2, _S, _DP), jnp.float32),
            pltpu.VMEM((2, _S, _D), jnp.float32),
            pltpu.SemaphoreType.DMA,
            pltpu.SemaphoreType.DMA,
            pltpu.SemaphoreType.DMA,
        ],
    )
    def k(x_hbm, table_hbm, pe_hbm, out_hbm, idx_v, pe_v, rows_v, sum_v,
          isem, gsem, psem):
        wid = lax.axis_index("s") * NC + lax.axis_index("c")
        base = wid * _ROWS_W
        pltpu.sync_copy(pe_hbm.at[pl.ds(0, _S)], pe_v)

        def idx_copy(kk, p):
            pltpu.async_copy(
                x_hbm.at[pl.ds((base + kk) * _S, _S)],
                idx_v.at[pl.ds(p * _S, _S)], isem)

        def chunk_body(kk, carry):
            p = kk % 2
            pltpu.make_async_copy(
                table_hbm.at[idx_v.at[pl.ds(0, _S)]], rows_v.at[p], gsem).wait()

            @pl.when(kk + 1 < _ROWS_W)
            def _():
                pltpu.make_async_copy(
                    x_hbm.at[pl.ds(0, _S)], idx_v.at[pl.ds(0, _S)], isem).wait()
                pltpu.async_copy(
                    table_hbm.at[idx_v.at[pl.ds((1 - p) * _S, _S)]],
                    rows_v.at[1 - p], gsem)

            @pl.when(kk + 2 < _ROWS_W)
            def _():
                idx_copy(kk + 2, p)

            @pl.when(kk >= 2)
            def _():
                pltpu.make_async_copy(
                    sum_v.at[0], out_hbm.at[base], psem).wait()

            rp = rows_v.at[p]
            sp = sum_v.at[p]

            def row_body(r, c2):
                for c in range(_D // L):
                    sl = pl.ds(c * L, L)
                    sp[r, sl] = rp[r, sl] + pe_v[r, sl]
                return c2

            lax.fori_loop(0, _S, row_body, 0)
            pltpu.async_copy(sp, out_hbm.at[base + kk], psem)
            return carry

        pltpu.sync_copy(x_hbm.at[pl.ds(base * _S, _S)], idx_v.at[pl.ds(0, _S)])
        pltpu.async_copy(
            table_hbm.at[idx_v.at[pl.ds(0, _S)]], rows_v.at[0], gsem)
        idx_copy(1, 1)
        lax.fori_loop(0, _ROWS_W, chunk_body, 0)
        for _ in range(2):
            pltpu.make_async_copy(sum_v.at[0], out_hbm.at[base], psem).wait()

    return k


_kernel_call = _make_kernel()


def kernel(x, token_table, pe):
    tab128 = jnp.pad(token_table, ((0, 0), (0, _D)))
    return _kernel_call(x.reshape(-1), tab128, pe)


# speedup vs baseline: 3.0629x; 1.0926x over previous
import jax
import jax.numpy as jnp
from jax import lax
from jax.experimental import pallas as pl
from jax.experimental.pallas import tpu as pltpu
from jax.experimental.pallas import tpu_sc as plsc

NC = 2
NS = 16
NW = NC * NS
L = 16

_B, _S, _D = 1024, 200, 64
_DP = 2 * _D
_ROWS_W = _B // NW


def _make_kernel():
    mesh = plsc.VectorSubcoreMesh(
        core_axis_name="c", subcore_axis_name="s",
        num_cores=NC, num_subcores=NS)

    @pl.kernel(
        out_type=jax.ShapeDtypeStruct((_B, _S, _D), jnp.float32),
        mesh=mesh,
        compiler_params=pltpu.CompilerParams(use_tc_tiling_on_sc=True),
        scratch_types=[
            pltpu.VMEM((2 * _S,), jnp.int32),
            pltpu.VMEM((_S, _D), jnp.float32),
            pltpu.VMEM((2, _S, _DP), jnp.float32),
            pltpu.VMEM((2, _S, _D), jnp.float32),
            pltpu.SemaphoreType.DMA,
            pltpu.SemaphoreType.DMA,
            pltpu.SemaphoreType.DMA,
        ],
    )
    def k(x_hbm, table_hbm, pe_hbm, out_hbm, idx_v, pe_v, rows_v, sum_v,
          isem, gsem, psem):
        wid = lax.axis_index("s") * NC + lax.axis_index("c")
        base = wid * _ROWS_W
        pltpu.sync_copy(pe_hbm.at[pl.ds(0, _S)], pe_v)

        def idx_copy(kk, p):
            pltpu.async_copy(
                x_hbm.at[pl.ds((base + kk) * _S, _S)],
                idx_v.at[pl.ds(p * _S, _S)], isem)

        def chunk_body(kk, carry):
            p = kk % 2
            pltpu.make_async_copy(
                table_hbm.at[idx_v.at[pl.ds(0, _S)]], rows_v.at[p], gsem).wait()

            @pl.when(kk + 1 < _ROWS_W)
            def _():
                pltpu.make_async_copy(
                    x_hbm.at[pl.ds(0, _S)], idx_v.at[pl.ds(0, _S)], isem).wait()
                pltpu.async_copy(
                    table_hbm.at[idx_v.at[pl.ds((1 - p) * _S, _S)]],
                    rows_v.at[1 - p], gsem)

            @pl.when(kk + 2 < _ROWS_W)
            def _():
                idx_copy(kk + 2, p)

            @pl.when(kk >= 2)
            def _():
                pltpu.make_async_copy(
                    sum_v.at[0], out_hbm.at[base], psem).wait()

            rp = rows_v.at[p]
            sp = sum_v.at[p]

            @plsc.parallel_loop(0, _S, unroll=8)
            def row_body(r):
                for c in range(_D // L):
                    sl = pl.ds(c * L, L)
                    sp[r, sl] = rp[r, sl] + pe_v[r, sl]
            pltpu.async_copy(sp, out_hbm.at[base + kk], psem)
            return carry

        pltpu.sync_copy(x_hbm.at[pl.ds(base * _S, _S)], idx_v.at[pl.ds(0, _S)])
        pltpu.async_copy(
            table_hbm.at[idx_v.at[pl.ds(0, _S)]], rows_v.at[0], gsem)
        idx_copy(1, 1)
        lax.fori_loop(0, _ROWS_W, chunk_body, 0)
        for _ in range(2):
            pltpu.make_async_copy(sum_v.at[0], out_hbm.at[base], psem).wait()

    return k


_kernel_call = _make_kernel()


def kernel(x, token_table, pe):
    tab128 = jnp.pad(token_table, ((0, 0), (0, _D)))
    return _kernel_call(x.reshape(-1), tab128, pe)
